# Initial kernel scaffold; baseline (speedup 1.0000x reference)
import jax
import jax.numpy as jnp
from jax.experimental import pallas as pl


def kernel(Z_img, Z_snd, snd_splits):
    raise NotImplementedError("write your pallas kernel here")


# speedup vs baseline: 3.2074x; 3.2074x over previous
import jax
import jax.numpy as jnp
from jax.experimental import pallas as pl
from jax.experimental.pallas import tpu as pltpu


def _img_mean_body(x_ref, o_ref):
    o_ref[...] = jnp.sum(x_ref[...], axis=2) * (1.0 / 196.0)


def _snd_sum_body(x_ref, o_ref):
    o_ref[...] = jnp.sum(x_ref[...], axis=1, keepdims=True)


def _bcast_body(inv_ref, img_ref, snd_ref, mimg_ref, msnd_ref):
    mimg_ref[...] = img_ref[...][None, :, :]
    row = snd_ref[...] * inv_ref[0]
    msnd_ref[...] = jnp.broadcast_to(row, msnd_ref.shape)


def kernel(Z_img, Z_snd, snd_splits):
    B, C, H, W = Z_img.shape
    HW = H * W
    S = 2048
    n_seg = Z_snd.shape[0] // S

    Z_img_flat = Z_img.reshape(B, C, HW)
    img_mean = pl.pallas_call(
        _img_mean_body,
        grid=(B // 8,),
        in_specs=[pl.BlockSpec((8, C, HW), lambda i: (i, 0, 0))],
        out_specs=pl.BlockSpec((8, C), lambda i: (i, 0)),
        out_shape=jax.ShapeDtypeStruct((B, C), jnp.float32),
    )(Z_img_flat)

    Z_snd_3d = Z_snd.reshape(n_seg, S, C)
    snd_sum = pl.pallas_call(
        _snd_sum_body,
        grid=(n_seg,),
        in_specs=[pl.BlockSpec((1, S, C), lambda i: (i, 0, 0))],
        out_specs=pl.BlockSpec((1, 1, C), lambda i: (i, 0, 0)),
        out_shape=jax.ShapeDtypeStruct((n_seg, 1, C), jnp.float32),
    )(Z_snd_3d)

    inv = (1.0 / snd_splits.astype(jnp.float32)).reshape(1)
    M_img, M_snd = pl.pallas_call(
        _bcast_body,
        grid=(n_seg,),
        in_specs=[
            pl.BlockSpec(memory_space=pltpu.SMEM),
            pl.BlockSpec((B, C), lambda i: (0, 0)),
            pl.BlockSpec((1, 1, C), lambda i: (i, 0, 0)),
        ],
        out_specs=[
            pl.BlockSpec((1, B, C), lambda i: (i, 0, 0)),
            pl.BlockSpec((1, B, C), lambda i: (i, 0, 0)),
        ],
        out_shape=[
            jax.ShapeDtypeStruct((n_seg, B, C), jnp.float32),
            jax.ShapeDtypeStruct((n_seg, B, C), jnp.float32),
        ],
    )(inv, img_mean, snd_sum)
    return (M_img, M_snd)
